# baseline (speedup 1.0000x reference)
import functools

import jax
import jax.numpy as jnp
from jax import lax
from jax.experimental import pallas as pl
from jax.experimental.pallas import tpu as pltpu
from jax.experimental.pallas import tpu_sc as plsc

NC = 2
NS = 16
GB = 128


NBUF = 2
SB = 4


@functools.lru_cache(maxsize=None)
def _build_sc_agg(n, ng, dp):
    assert n % (8 * NS) == 0 and ng % SB == 0 and (ng // SB) % 2 == 0
    rpt = n // NS
    nsb = ng // SB
    mesh = plsc.VectorSubcoreMesh(core_axis_name="c", subcore_axis_name="s")

    @functools.partial(
        pl.kernel,
        mesh=mesh,
        compiler_params=pltpu.CompilerParams(use_tc_tiling_on_sc=False),
        out_type=jax.ShapeDtypeStruct((NC, n, dp), jnp.float32),
        scratch_types=[
            pltpu.VMEM((2, SB, GB), jnp.int32),
            pltpu.VMEM((2, SB, GB), jnp.int32),
            pltpu.VMEM((NBUF, GB, dp), jnp.float32),
            pltpu.VMEM_SHARED((n, dp), jnp.float32),
            pltpu.SemaphoreType.DMA((NBUF,)),
            pltpu.SemaphoreType.DMA((NBUF,)),
            pltpu.SemaphoreType.DMA((2,)),
        ],
    )
    def sc_agg(feat_hbm, src_hbm, dst_hbm, zero_hbm, out_hbm,
               sidx, didx, rows, acc, gsem, ssem, isem):
        c = lax.axis_index("c")
        s = lax.axis_index("s")
        wid = s * NC + c
        base_g = wid * ng

        def idx_load(sb, buf):
            pltpu.async_copy(src_hbm.at[pl.ds(base_g + sb * SB, SB)],
                             sidx.at[buf], isem.at[buf])
            pltpu.async_copy(dst_hbm.at[pl.ds(base_g + sb * SB, SB)],
                             didx.at[buf], isem.at[buf])

        def idx_wait(buf):
            pltpu.make_async_copy(src_hbm.at[pl.ds(base_g, SB)],
                                  sidx.at[buf], isem.at[buf]).wait()
            pltpu.make_async_copy(dst_hbm.at[pl.ds(base_g, SB)],
                                  didx.at[buf], isem.at[buf]).wait()

        idx_load(0, 0)
        pltpu.sync_copy(zero_hbm.at[pl.ds(s * rpt, rpt)],
                        acc.at[pl.ds(s * rpt, rpt)])
        idx_wait(0)
        for b in range(NBUF):
            pltpu.async_copy(feat_hbm.at[sidx.at[0, b]], rows.at[b],
                             gsem.at[b])
        idx_load(1, 1)
        plsc.subcore_barrier()

        def outer(sb, carry):
            ib = lax.rem(sb, 2)
            for k in range(SB):
                b = k % NBUF
                pltpu.make_async_copy(feat_hbm.at[sidx.at[ib, k]],
                                      rows.at[b], gsem.at[b]).wait()
                pltpu.async_copy(rows.at[b], acc.at[didx.at[ib, k]],
                                 ssem.at[b], add=True)
                if k < SB - NBUF:
                    pltpu.make_async_copy(rows.at[b], acc.at[didx.at[ib, k]],
                                          ssem.at[b]).wait()
                    pltpu.async_copy(feat_hbm.at[sidx.at[ib, k + NBUF]],
                                     rows.at[b], gsem.at[b])
                else:
                    kk = k - (SB - NBUF)

                    @pl.when(sb + 1 < nsb)
                    def _():
                        if kk == 0:
                            idx_wait(1 - ib)
                        pltpu.make_async_copy(rows.at[b],
                                              acc.at[didx.at[ib, k]],
                                              ssem.at[b]).wait()
                        pltpu.async_copy(feat_hbm.at[sidx.at[1 - ib, kk]],
                                         rows.at[b], gsem.at[b])

            @pl.when(sb + 2 < nsb)
            def _():
                idx_load(sb + 2, ib)

            return carry

        lax.fori_loop(0, nsb, outer, 0)
        for b in range(NBUF):
            pltpu.make_async_copy(rows.at[b], acc.at[didx.at[0, b]],
                                  ssem.at[b]).wait()
        plsc.subcore_barrier()
        pltpu.sync_copy(acc.at[pl.ds(s * rpt, rpt)],
                        out_hbm.at[c, pl.ds(s * rpt, rpt)])

    return sc_agg


def _tc_body(p_ref, feat_ref, w_ref, b_ref, out_ref):
    acc = p_ref[0] + p_ref[1]
    agg = acc[:, :128]
    deg = jnp.sum(acc[:, 128:144], axis=1, keepdims=True)
    h = (agg + feat_ref[...]) / (deg + 1.0)
    r = lax.dot_general(h, w_ref[...], (((1,), (1,)), ((), ())),
                        preferred_element_type=jnp.float32)
    r = r + b_ref[...]
    out_ref[...] = jnp.where(r >= 0, r, 0.01 * r)


def kernel(feat, edge_index, W_neigh, b_neigh):
    n, d = feat.shape
    e = edge_index.shape[1]
    dp = d + 16
    npad = -(-n // 128) * 128
    ng = -(-(-(-e // GB) // (NC * NS)) // (2 * SB)) * (2 * SB)
    e_pad = ng * NC * NS * GB
    src = jnp.concatenate(
        [edge_index[0], jnp.zeros((e_pad - e,), jnp.int32)])
    dst = jnp.concatenate(
        [edge_index[1], jnp.full((e_pad - e,), npad - 1, jnp.int32)])
    src2d = src.reshape(e_pad // GB, GB)
    dst2d = dst.reshape(e_pad // GB, GB)
    feat_pad = jnp.concatenate(
        [feat,
         jnp.ones((n, 1), jnp.float32),
         jnp.zeros((n, 15), jnp.float32)], axis=1)
    zero_init = jnp.zeros((npad, dp), jnp.float32)

    partials = _build_sc_agg(npad, ng, dp)(feat_pad, src2d, dst2d, zero_init)

    bn = 1000 if n % 1000 == 0 else n
    grid = n // bn
    out = pl.pallas_call(
        _tc_body,
        grid=(grid,),
        in_specs=[
            pl.BlockSpec((NC, bn, dp), lambda i: (0, i, 0)),
            pl.BlockSpec((bn, d), lambda i: (i, 0)),
            pl.BlockSpec(W_neigh.shape, lambda i: (0, 0)),
            pl.BlockSpec((1, b_neigh.shape[0]), lambda i: (0, 0)),
        ],
        out_specs=pl.BlockSpec((bn, d), lambda i: (i, 0)),
        out_shape=jax.ShapeDtypeStruct((n, d), jnp.float32),
    )(partials, feat, W_neigh, b_neigh.reshape(1, -1))
    return out
